# baseline (device time: 140593 ns/iter reference)
import jax
import jax.numpy as jnp
from jax import lax
from jax.experimental import pallas as pl
from jax.experimental.pallas import tpu as pltpu

N_DEV = 16


def kernel(x, dy):
    m, d = x.shape
    _, f = dy.shape
    chunk = d // N_DEV

    def body(x_ref, dy_ref, out_ref,
             xb_ref, dyb_ref, partial_ref, send_ref, comm_ref,
             send_sems, recv_sems):
        my = lax.axis_index("i")
        right = lax.rem(my + 1, N_DEV)

        xb_ref[...] = x_ref[...].astype(jnp.bfloat16)
        dyb_ref[...] = dy_ref[...].astype(jnp.bfloat16)
        partial_ref[...] = lax.dot_general(
            xb_ref[...], dyb_ref[...],
            dimension_numbers=(((0,), (0,)), ((), ())),
            preferred_element_type=jnp.float32,
        )

        def partial_chunk(c):
            return partial_ref[pl.ds(c * chunk, chunk), :]

        c0 = lax.rem(my + N_DEV - 1, N_DEV)
        send_ref[...] = partial_chunk(c0).astype(jnp.bfloat16)

        for s in range(N_DEV - 1):
            rdma = pltpu.make_async_remote_copy(
                src_ref=send_ref,
                dst_ref=comm_ref.at[s],
                send_sem=send_sems.at[s],
                recv_sem=recv_sems.at[s],
                device_id=(right,),
                device_id_type=pl.DeviceIdType.MESH,
            )
            rdma.start()
            rdma.wait()
            c = lax.rem(my + 2 * N_DEV - 2 - s, N_DEV)
            acc = comm_ref[s].astype(jnp.float32) + partial_chunk(c)
            if s < N_DEV - 2:
                send_ref[...] = acc.astype(jnp.bfloat16)
            else:
                out_ref[...] = acc

    return pl.pallas_call(
        body,
        out_shape=jax.ShapeDtypeStruct((chunk, f), jnp.float32),
        in_specs=[
            pl.BlockSpec(memory_space=pltpu.VMEM),
            pl.BlockSpec(memory_space=pltpu.VMEM),
        ],
        out_specs=pl.BlockSpec(memory_space=pltpu.VMEM),
        scratch_shapes=[
            pltpu.VMEM((m, d), jnp.bfloat16),
            pltpu.VMEM((m, f), jnp.bfloat16),
            pltpu.VMEM((d, f), jnp.float32),
            pltpu.VMEM((chunk, f), jnp.bfloat16),
            pltpu.VMEM((N_DEV - 1, chunk, f), jnp.bfloat16),
            pltpu.SemaphoreType.DMA((N_DEV - 1,)),
            pltpu.SemaphoreType.DMA((N_DEV - 1,)),
        ],
    )(x, dy)


# device time: 80046 ns/iter; 1.7564x vs baseline; 1.7564x over previous
import jax
import jax.numpy as jnp
from jax import lax
from jax.experimental import pallas as pl
from jax.experimental.pallas import tpu as pltpu

N_DEV = 16
S = 4


def kernel(x, dy):
    m, d = x.shape
    _, f = dy.shape
    chunk = d // N_DEV
    fs = f // S

    def body(x_ref, dy_ref, out_ref,
             xb_ref, partial_ref, send_ref, comm_ref,
             send_sems, recv_sems):
        my = lax.axis_index("i")
        right = lax.rem(my + 1, N_DEV)
        left = lax.rem(my + N_DEV - 1, N_DEV)

        barrier_sem = pltpu.get_barrier_semaphore()
        for nbr in (left, right):
            pl.semaphore_signal(
                barrier_sem, inc=1,
                device_id=(nbr,), device_id_type=pl.DeviceIdType.MESH,
            )
        pl.semaphore_wait(barrier_sem, 2)

        xb_ref[...] = x_ref[...].astype(jnp.bfloat16)

        def fcols(j):
            return slice(j * fs, (j + 1) * fs)

        def c_send(j, s):
            if j % 2 == 0:
                return lax.rem(my + 2 * N_DEV - 1 - s, N_DEV)
            return lax.rem(my + 1 + s, N_DEV)

        def prow(c):
            return pl.ds(c * chunk, chunk)

        def make_rdma(j, s):
            return pltpu.make_async_remote_copy(
                src_ref=send_ref.at[j, s % 2],
                dst_ref=comm_ref.at[j, s],
                send_sem=send_sems.at[j, s % 2],
                recv_sem=recv_sems.at[j, s],
                device_id=(right if j % 2 == 0 else left,),
                device_id_type=pl.DeviceIdType.MESH,
            )

        for j in range(S):
            partial_ref[:, fcols(j)] = lax.dot_general(
                xb_ref[...], dy_ref[:, fcols(j)].astype(jnp.bfloat16),
                dimension_numbers=(((0,), (0,)), ((), ())),
                preferred_element_type=jnp.float32,
            ).astype(jnp.bfloat16)
            send_ref[j, 0] = partial_ref[prow(c_send(j, 0)), fcols(j)]
            make_rdma(j, 0).start()

        for s in range(N_DEV - 1):
            for j in range(S):
                make_rdma(j, s).wait_recv()
                c = c_send(j, s + 1)
                acc = (comm_ref[j, s].astype(jnp.float32)
                       + partial_ref[prow(c), fcols(j)].astype(jnp.float32))
                if s < N_DEV - 2:
                    if s >= 1:
                        make_rdma(j, s - 1).wait_send()
                    send_ref[j, (s + 1) % 2] = acc.astype(jnp.bfloat16)
                    make_rdma(j, s + 1).start()
                else:
                    out_ref[:, fcols(j)] = acc

        for s in (N_DEV - 3, N_DEV - 2):
            for j in range(S):
                make_rdma(j, s).wait_send()

    nh = N_DEV - 1
    return pl.pallas_call(
        body,
        out_shape=jax.ShapeDtypeStruct((chunk, f), jnp.float32),
        in_specs=[
            pl.BlockSpec(memory_space=pltpu.VMEM),
            pl.BlockSpec(memory_space=pltpu.VMEM),
        ],
        out_specs=pl.BlockSpec(memory_space=pltpu.VMEM),
        scratch_shapes=[
            pltpu.VMEM((m, d), jnp.bfloat16),
            pltpu.VMEM((d, f), jnp.bfloat16),
            pltpu.VMEM((S, 2, chunk, fs), jnp.bfloat16),
            pltpu.VMEM((S, nh, chunk, fs), jnp.bfloat16),
            pltpu.SemaphoreType.DMA((S, 2)),
            pltpu.SemaphoreType.DMA((S, nh)),
        ],
        compiler_params=pltpu.CompilerParams(collective_id=0),
    )(x, dy)
